# baseline (speedup 1.0000x reference)
import functools

import jax
import jax.numpy as jnp
from jax import lax
from jax.experimental import pallas as pl
from jax.experimental.pallas import tpu as pltpu
from jax.experimental.pallas import tpu_sc as plsc

N = 100000
E = 3200000
EP = 3276800
ER = EP // 128
ALPHA = 0.9
KR = 8
NP = 100352
TPT = NP // 16
ZROWS = 112
ZREP = TPT // ZROWS


def _make_spmm(split_features: bool, xt_rows: int):
    mesh = plsc.VectorSubcoreMesh(core_axis_name="c", subcore_axis_name="s")
    DEPTH = 6

    @functools.partial(
        pl.kernel,
        out_type=jax.ShapeDtypeStruct((2, NP, 16), jnp.float32),
        mesh=mesh,
        compiler_params=pltpu.CompilerParams(use_tc_tiling_on_sc=False),
        scratch_types=[
            pltpu.VMEM((4, KR, 128), jnp.int32),
            pltpu.VMEM((4, KR, 128), jnp.int32),
            pltpu.VMEM((4, KR, 128), jnp.float32),
            pltpu.VMEM((KR * 128, 16), jnp.float32),
            pltpu.VMEM((ZROWS, 16), jnp.float32),
            pltpu.VMEM_SHARED((NP, 16), jnp.float32),
            pltpu.SemaphoreType.DMA((4,)),
            pltpu.SemaphoreType.DMA((KR,)),
            pltpu.SemaphoreType.DMA((KR,)),
        ],
    )
    def spmm(xt, srcr, dstr, valr, out, sbuf, dbuf, vbuf, rows, zbuf, acc,
             isem, gsem, ssem):
        c = lax.axis_index("c")
        s = lax.axis_index("s")

        with jax.named_scope("acc_zero"):
            zeros16 = jnp.zeros((16,), jnp.float32)

            def zb(i, carry):
                zbuf[i, :] = zeros16
                return carry

            lax.fori_loop(0, ZROWS, zb, 0)
            zb0 = s * TPT

            def zcopy(z, carry):
                pltpu.sync_copy(zbuf, acc.at[pl.ds(zb0 + z * ZROWS, ZROWS)])
                return carry

            lax.fori_loop(0, ZREP, zcopy, 0)
            plsc.subcore_barrier()

        if split_features:
            sbase = c * ER + s * (ER // 16)
            dbase = s * (ER // 16)
            ngroups = ER // 16
        else:
            w = s * 2 + c
            sbase = w * (ER // 32)
            dbase = sbase
            ngroups = ER // 32
        nmac = ngroups // KR

        def stage_idx(k, slot):
            pltpu.async_copy(srcr.at[pl.ds(sbase + k * KR, KR)],
                             sbuf.at[slot], isem.at[slot])
            pltpu.async_copy(dstr.at[pl.ds(dbase + k * KR, KR)],
                             dbuf.at[slot], isem.at[slot])
            pltpu.async_copy(valr.at[pl.ds(dbase + k * KR, KR)],
                             vbuf.at[slot], isem.at[slot])

        def wait_idx(k, slot):
            pltpu.make_async_copy(srcr.at[pl.ds(sbase + k * KR, KR)],
                                  sbuf.at[slot], isem.at[slot]).wait()
            pltpu.make_async_copy(dstr.at[pl.ds(dbase + k * KR, KR)],
                                  dbuf.at[slot], isem.at[slot]).wait()
            pltpu.make_async_copy(valr.at[pl.ds(dbase + k * KR, KR)],
                                  vbuf.at[slot], isem.at[slot]).wait()

        def fire_gather(f):
            kf = lax.rem(lax.div(f, KR), 4)
            fr = lax.rem(f, KR)
            pltpu.async_copy(xt.at[sbuf.at[kf, fr]],
                             rows.at[pl.ds(fr * 128, 128)], gsem.at[fr])

        stage_idx(0, 0)
        stage_idx(1, 1)
        stage_idx(2, 2)
        wait_idx(0, 0)
        for g in range(DEPTH):
            fire_gather(g)

        def step(g, carry):
            r = lax.rem(g, KR)
            km = lax.rem(lax.div(g, KR), 4)
            rj = rows.at[pl.ds(r * 128, 128)]
            pltpu.make_async_copy(xt.at[sbuf.at[km, r]], rj,
                                  gsem.at[r]).wait()
            for q in range(8):
                vch = vbuf[km, r, pl.ds(q * 16, 16)]
                for l in range(16):
                    b = lax.gather(
                        vch, jnp.full((16, 1), l, jnp.int32),
                        lax.GatherDimensionNumbers(
                            offset_dims=(), collapsed_slice_dims=(0,),
                            start_index_map=(0,)),
                        (1,), mode=lax.GatherScatterMode.PROMISE_IN_BOUNDS)
                    rr = r * 128 + q * 16 + l
                    rows[rr, :] = rows[rr, :] * b
            pltpu.async_copy(rj, acc.at[dbuf.at[km, r]], ssem.at[r],
                             add=True)

            f = g + DEPTH

            @pl.when(f < ngroups)
            def _ahead():
                kf = lax.div(f, KR)
                fr = lax.rem(f, KR)

                @pl.when(lax.rem(f, KR) == 0)
                def _boundary():
                    @pl.when(kf + 2 < nmac)
                    def _stage():
                        stage_idx(kf + 2, lax.rem(kf + 2, 4))

                    wait_idx(kf, lax.rem(kf, 4))

                @pl.when(f >= KR)
                def _wait_prev_scatter():
                    kp = lax.rem(lax.div(f, KR) + 3, 4)
                    pltpu.make_async_copy(
                        rows.at[pl.ds(fr * 128, 128)],
                        acc.at[dbuf.at[kp, fr]], ssem.at[fr]).wait()

                fire_gather(f)

            return carry

        with jax.named_scope("edge_loop"):
            lax.fori_loop(0, ngroups, step, 0)

            klast = (nmac - 1) % 4
            for g in range(KR):
                pltpu.make_async_copy(rows.at[pl.ds(g * 128, 128)],
                                      acc.at[dbuf.at[klast, g]],
                                      ssem.at[g]).wait()

        with jax.named_scope("writeback"):
            plsc.subcore_barrier()
            wb0 = s * TPT
            pltpu.sync_copy(acc.at[pl.ds(wb0, TPT)],
                            out.at[c, pl.ds(wb0, TPT)])

    return spmm


_spmm16 = _make_spmm(split_features=False, xt_rows=N)
_spmm32 = _make_spmm(split_features=True, xt_rows=2 * N)

_B = 2000
_NB = N // _B
_PREC = lax.Precision.HIGHEST


def _dot(a, b):
    return jnp.dot(a, b, preferred_element_type=jnp.float32, precision=_PREC)


def _transform0(g, x0p, wi0p, wx0p):
    def body(g0, g1, x0, wi, wx, out):
        u = ALPHA * (g0[0] + g1[0]) + (1.0 - ALPHA) * _dot(x0[...], wi[...])
        x1 = jnp.maximum(_dot(u, wx[...]), 0.0)
        out[0, :, :] = x1[:, :16]
        out[1, :, :] = x1[:, 16:]

    return pl.pallas_call(
        body,
        grid=(_NB,),
        in_specs=[
            pl.BlockSpec((1, _B, 16), lambda i: (0, i, 0)),
            pl.BlockSpec((1, _B, 16), lambda i: (1, i, 0)),
            pl.BlockSpec((_B, 16), lambda i: (i, 0)),
            pl.BlockSpec((16, 16), lambda i: (0, 0)),
            pl.BlockSpec((16, 32), lambda i: (0, 0)),
        ],
        out_specs=pl.BlockSpec((2, _B, 16), lambda i: (0, i, 0)),
        out_shape=jax.ShapeDtypeStruct((2, N, 16), jnp.float32),
    )(g, g, x0p, wi0p, wx0p)


def _transform_mid(g, x0p, wip, wx):
    def body(g0, g1, x0, wi, wxr, out):
        gcat = jnp.concatenate([g0[0], g1[0]], axis=1)
        u = ALPHA * gcat + (1.0 - ALPHA) * _dot(x0[...], wi[...])
        x = jnp.maximum(_dot(u, wxr[...]), 0.0)
        out[0, :, :] = x[:, :16]
        out[1, :, :] = x[:, 16:]

    return pl.pallas_call(
        body,
        grid=(_NB,),
        in_specs=[
            pl.BlockSpec((1, _B, 16), lambda i: (0, i, 0)),
            pl.BlockSpec((1, _B, 16), lambda i: (1, i, 0)),
            pl.BlockSpec((_B, 16), lambda i: (i, 0)),
            pl.BlockSpec((16, 32), lambda i: (0, 0)),
            pl.BlockSpec((32, 32), lambda i: (0, 0)),
        ],
        out_specs=pl.BlockSpec((2, _B, 16), lambda i: (0, i, 0)),
        out_shape=jax.ShapeDtypeStruct((2, N, 16), jnp.float32),
    )(g, g, x0p, wip, wx)


def _transform_y(g, x0p, wip, wx, wx3p):
    def body(g0, g1, x0, wi, wxr, wx3, out):
        gcat = jnp.concatenate([g0[0], g1[0]], axis=1)
        u = ALPHA * gcat + (1.0 - ALPHA) * _dot(x0[...], wi[...])
        x = jnp.maximum(_dot(u, wxr[...]), 0.0)
        out[...] = _dot(x, wx3[...])

    return pl.pallas_call(
        body,
        grid=(_NB,),
        in_specs=[
            pl.BlockSpec((1, _B, 16), lambda i: (0, i, 0)),
            pl.BlockSpec((1, _B, 16), lambda i: (1, i, 0)),
            pl.BlockSpec((_B, 16), lambda i: (i, 0)),
            pl.BlockSpec((16, 32), lambda i: (0, 0)),
            pl.BlockSpec((32, 32), lambda i: (0, 0)),
            pl.BlockSpec((32, 16), lambda i: (0, 0)),
        ],
        out_specs=pl.BlockSpec((_B, 16), lambda i: (i, 0)),
        out_shape=jax.ShapeDtypeStruct((N, 16), jnp.float32),
    )(g, g, x0p, wip, wx, wx3p)


def _transform_last(g, x0p, wi3p, wx3p):
    def body(g0, g1, x0, wi, wx, out):
        c3 = _dot(wi[...], wx[...])
        t = _dot(x0[...], c3)
        v = ALPHA * (g0[0] + g1[0]) + (1.0 - ALPHA) * t
        vc = v[:, 0:1]
        out[...] = 1.0 / (1.0 + jnp.exp(-vc))

    return pl.pallas_call(
        body,
        grid=(_NB,),
        in_specs=[
            pl.BlockSpec((1, _B, 16), lambda i: (0, i, 0)),
            pl.BlockSpec((1, _B, 16), lambda i: (1, i, 0)),
            pl.BlockSpec((_B, 16), lambda i: (i, 0)),
            pl.BlockSpec((16, 32), lambda i: (0, 0)),
            pl.BlockSpec((32, 16), lambda i: (0, 0)),
        ],
        out_specs=pl.BlockSpec((_B, 1), lambda i: (i, 0)),
        out_shape=jax.ShapeDtypeStruct((N, 1), jnp.float32),
    )(g, g, x0p, wi3p, wx3p)


def kernel(inputs, edge_index, adj_vals, w_init_0, w_x_0, w_init_1, w_x_1,
           w_init_2, w_x_2, w_init_3, w_x_3):
    src = edge_index[0]
    dst = edge_index[1]
    pad = EP - E
    srcp = jnp.concatenate([src, jnp.zeros((pad,), jnp.int32)])
    dstp = jnp.concatenate([dst, jnp.zeros((pad,), jnp.int32)])
    valp = jnp.concatenate([adj_vals, jnp.zeros((pad,), jnp.float32)])
    src2 = jnp.concatenate([srcp, srcp + N]).reshape(2 * ER, 128)
    dst2 = dstp.reshape(ER, 128)
    val2 = valp.reshape(ER, 128)

    x0p = jnp.pad(inputs, ((0, 0), (0, 13)))
    wi0p = jnp.pad(w_init_0, ((0, 13), (0, 13)))
    wx0p = jnp.pad(w_x_0, ((0, 13), (0, 0)))
    wi1p = jnp.pad(w_init_1, ((0, 13), (0, 0)))
    wi2p = jnp.pad(w_init_2, ((0, 13), (0, 0)))
    wi3p = jnp.pad(w_init_3, ((0, 13), (0, 0)))
    wx3p = jnp.pad(w_x_3, ((0, 0), (0, 15)))

    g0 = _spmm16(x0p, src2, dst2, val2)
    x1 = _transform0(g0, x0p, wi0p, wx0p).reshape(2 * N, 16)
    g1 = _spmm32(x1, src2, dst2, val2)
    x2 = _transform_mid(g1, x0p, wi1p, w_x_1).reshape(2 * N, 16)
    g2 = _spmm32(x2, src2, dst2, val2)
    y3 = _transform_y(g2, x0p, wi2p, w_x_2, wx3p)
    g3 = _spmm16(y3, src2, dst2, val2)
    return _transform_last(g3, x0p, wi3p, wx3p)


# speedup vs baseline: 1.9718x; 1.9718x over previous
import functools

import jax
import jax.numpy as jnp
from jax import lax
from jax.experimental import pallas as pl
from jax.experimental.pallas import tpu as pltpu
from jax.experimental.pallas import tpu_sc as plsc

N = 100000
E = 3200000
EP = 3276800
ER = EP // 128
ALPHA = 0.9
KR = 8
NP = 100352
TPT = NP // 16
ZROWS = 112
ZREP = TPT // ZROWS


def _make_spmm(split_features: bool, xt_rows: int):
    mesh = plsc.VectorSubcoreMesh(core_axis_name="c", subcore_axis_name="s")
    DEPTH = 6

    @functools.partial(
        pl.kernel,
        out_type=jax.ShapeDtypeStruct((2, NP, 16), jnp.float32),
        mesh=mesh,
        compiler_params=pltpu.CompilerParams(use_tc_tiling_on_sc=False),
        scratch_types=[
            pltpu.VMEM((4, KR, 128), jnp.int32),
            pltpu.VMEM((4, KR, 128), jnp.int32),
            pltpu.VMEM((4, KR, 128), jnp.float32),
            pltpu.VMEM((KR * 128, 16), jnp.float32),
            pltpu.VMEM((ZROWS, 16), jnp.float32),
            pltpu.VMEM_SHARED((NP, 16), jnp.float32),
            pltpu.SemaphoreType.DMA((4,)),
            pltpu.SemaphoreType.DMA((KR,)),
            pltpu.SemaphoreType.DMA((KR,)),
        ],
    )
    def spmm(xt, srcr, dstr, valr, out, sbuf, dbuf, vbuf, rows, zbuf, acc,
             isem, gsem, ssem):
        c = lax.axis_index("c")
        s = lax.axis_index("s")

        with jax.named_scope("acc_zero"):
            zeros16 = jnp.zeros((16,), jnp.float32)

            def zb(i, carry):
                zbuf[i, :] = zeros16
                return carry

            lax.fori_loop(0, ZROWS, zb, 0)
            zb0 = s * TPT

            def zcopy(z, carry):
                pltpu.sync_copy(zbuf, acc.at[pl.ds(zb0 + z * ZROWS, ZROWS)])
                return carry

            lax.fori_loop(0, ZREP, zcopy, 0)
            plsc.subcore_barrier()

        if split_features:
            sbase = c * ER + s * (ER // 16)
            dbase = s * (ER // 16)
            ngroups = ER // 16
        else:
            w = s * 2 + c
            sbase = w * (ER // 32)
            dbase = sbase
            ngroups = ER // 32
        nmac = ngroups // KR

        def stage_idx(k, slot):
            pltpu.async_copy(srcr.at[pl.ds(sbase + k * KR, KR)],
                             sbuf.at[slot], isem.at[slot])
            pltpu.async_copy(dstr.at[pl.ds(dbase + k * KR, KR)],
                             dbuf.at[slot], isem.at[slot])
            pltpu.async_copy(valr.at[pl.ds(dbase + k * KR, KR)],
                             vbuf.at[slot], isem.at[slot])

        def wait_idx(k, slot):
            pltpu.make_async_copy(srcr.at[pl.ds(sbase + k * KR, KR)],
                                  sbuf.at[slot], isem.at[slot]).wait()
            pltpu.make_async_copy(dstr.at[pl.ds(dbase + k * KR, KR)],
                                  dbuf.at[slot], isem.at[slot]).wait()
            pltpu.make_async_copy(valr.at[pl.ds(dbase + k * KR, KR)],
                                  vbuf.at[slot], isem.at[slot]).wait()

        def fire_gather(f):
            kf = lax.rem(lax.div(f, KR), 4)
            fr = lax.rem(f, KR)
            pltpu.async_copy(xt.at[sbuf.at[kf, fr]],
                             rows.at[pl.ds(fr * 128, 128)], gsem.at[fr])

        stage_idx(0, 0)
        stage_idx(1, 1)
        stage_idx(2, 2)
        wait_idx(0, 0)
        for g in range(DEPTH):
            fire_gather(g)

        def step(g, carry):
            r = lax.rem(g, KR)
            km = lax.rem(lax.div(g, KR), 4)
            rj = rows.at[pl.ds(r * 128, 128)]
            pltpu.make_async_copy(xt.at[sbuf.at[km, r]], rj,
                                  gsem.at[r]).wait()
            for q in range(8):
                vch = vbuf[km, r, pl.ds(q * 16, 16)]
                for l in range(16):
                    b = lax.gather(
                        vch, jnp.full((16, 1), l, jnp.int32),
                        lax.GatherDimensionNumbers(
                            offset_dims=(), collapsed_slice_dims=(0,),
                            start_index_map=(0,)),
                        (1,), mode=lax.GatherScatterMode.PROMISE_IN_BOUNDS)
                    rr = r * 128 + q * 16 + l
                    rows[rr, :] = rows[rr, :] * b
            pltpu.async_copy(rj, acc.at[dbuf.at[km, r]], ssem.at[r],
                             add=True)

            f = g + DEPTH

            @pl.when(f < ngroups)
            def _ahead():
                kf = lax.div(f, KR)
                fr = lax.rem(f, KR)

                @pl.when(lax.rem(f, KR) == 0)
                def _boundary():
                    @pl.when(kf + 2 < nmac)
                    def _stage():
                        stage_idx(kf + 2, lax.rem(kf + 2, 4))

                    wait_idx(kf, lax.rem(kf, 4))

                @pl.when(f >= KR)
                def _wait_prev_scatter():
                    kp = lax.rem(lax.div(f, KR) + 3, 4)
                    pltpu.make_async_copy(
                        rows.at[pl.ds(fr * 128, 128)],
                        acc.at[dbuf.at[kp, fr]], ssem.at[fr]).wait()

                fire_gather(f)

            return carry

        with jax.named_scope("edge_loop"):
            lax.fori_loop(0, ngroups, step, 0)

            klast = (nmac - 1) % 4
            for g in range(KR):
                pltpu.make_async_copy(rows.at[pl.ds(g * 128, 128)],
                                      acc.at[dbuf.at[klast, g]],
                                      ssem.at[g]).wait()

        with jax.named_scope("writeback"):
            plsc.subcore_barrier()
            wb0 = s * TPT
            pltpu.sync_copy(acc.at[pl.ds(wb0, TPT)],
                            out.at[c, pl.ds(wb0, TPT)])

    return spmm


_spmm16 = _make_spmm(split_features=False, xt_rows=N)
_spmm32 = _make_spmm(split_features=True, xt_rows=2 * N)

_B = 2000
_NB = N // _B
_PREC = lax.Precision.HIGHEST


def _dot(a, b):
    return jnp.dot(a, b, preferred_element_type=jnp.float32, precision=_PREC)


def _transform0(g, x0p, wi0p, wx0p):
    def body(g0, g1, x0, wi, wx, out):
        u = ALPHA * (g0[0] + g1[0]) + (1.0 - ALPHA) * _dot(x0[...], wi[...])
        x1 = jnp.maximum(_dot(u, wx[...]), 0.0)
        out[0, :, :] = x1[:, :16]
        out[1, :, :] = x1[:, 16:]

    return pl.pallas_call(
        body,
        grid=(_NB,),
        in_specs=[
            pl.BlockSpec((1, _B, 16), lambda i: (0, i, 0)),
            pl.BlockSpec((1, _B, 16), lambda i: (1, i, 0)),
            pl.BlockSpec((_B, 16), lambda i: (i, 0)),
            pl.BlockSpec((16, 16), lambda i: (0, 0)),
            pl.BlockSpec((16, 32), lambda i: (0, 0)),
        ],
        out_specs=pl.BlockSpec((2, _B, 16), lambda i: (0, i, 0)),
        out_shape=jax.ShapeDtypeStruct((2, N, 16), jnp.float32),
    )(g, g, x0p, wi0p, wx0p)


def _transform_mid(g, x0p, wip, wx):
    def body(g0, g1, x0, wi, wxr, out):
        gcat = jnp.concatenate([g0[0], g1[0]], axis=1)
        u = ALPHA * gcat + (1.0 - ALPHA) * _dot(x0[...], wi[...])
        x = jnp.maximum(_dot(u, wxr[...]), 0.0)
        out[0, :, :] = x[:, :16]
        out[1, :, :] = x[:, 16:]

    return pl.pallas_call(
        body,
        grid=(_NB,),
        in_specs=[
            pl.BlockSpec((1, _B, 16), lambda i: (0, i, 0)),
            pl.BlockSpec((1, _B, 16), lambda i: (1, i, 0)),
            pl.BlockSpec((_B, 16), lambda i: (i, 0)),
            pl.BlockSpec((16, 32), lambda i: (0, 0)),
            pl.BlockSpec((32, 32), lambda i: (0, 0)),
        ],
        out_specs=pl.BlockSpec((2, _B, 16), lambda i: (0, i, 0)),
        out_shape=jax.ShapeDtypeStruct((2, N, 16), jnp.float32),
    )(g, g, x0p, wip, wx)


def _transform_y(g, x0p, wip, wx, wx3p):
    def body(g0, g1, x0, wi, wxr, wx3, out):
        gcat = jnp.concatenate([g0[0], g1[0]], axis=1)
        u = ALPHA * gcat + (1.0 - ALPHA) * _dot(x0[...], wi[...])
        x = jnp.maximum(_dot(u, wxr[...]), 0.0)
        out[...] = _dot(x, wx3[...])

    return pl.pallas_call(
        body,
        grid=(_NB,),
        in_specs=[
            pl.BlockSpec((1, _B, 16), lambda i: (0, i, 0)),
            pl.BlockSpec((1, _B, 16), lambda i: (1, i, 0)),
            pl.BlockSpec((_B, 16), lambda i: (i, 0)),
            pl.BlockSpec((16, 32), lambda i: (0, 0)),
            pl.BlockSpec((32, 32), lambda i: (0, 0)),
            pl.BlockSpec((32, 16), lambda i: (0, 0)),
        ],
        out_specs=pl.BlockSpec((_B, 16), lambda i: (i, 0)),
        out_shape=jax.ShapeDtypeStruct((N, 16), jnp.float32),
    )(g, g, x0p, wip, wx, wx3p)


def _transform_last(g, x0p, wi3p, wx3p):
    def body(g0, g1, x0, wi, wx, out):
        c3 = _dot(wi[...], wx[...])
        t = _dot(x0[...], c3)
        v = ALPHA * (g0[0] + g1[0]) + (1.0 - ALPHA) * t
        vc = v[:, 0:1]
        out[...] = 1.0 / (1.0 + jnp.exp(-vc))

    return pl.pallas_call(
        body,
        grid=(_NB,),
        in_specs=[
            pl.BlockSpec((1, _B, 16), lambda i: (0, i, 0)),
            pl.BlockSpec((1, _B, 16), lambda i: (1, i, 0)),
            pl.BlockSpec((_B, 16), lambda i: (i, 0)),
            pl.BlockSpec((16, 32), lambda i: (0, 0)),
            pl.BlockSpec((32, 16), lambda i: (0, 0)),
        ],
        out_specs=pl.BlockSpec((_B, 1), lambda i: (i, 0)),
        out_shape=jax.ShapeDtypeStruct((N, 1), jnp.float32),
    )(g, g, x0p, wi3p, wx3p)


def kernel(inputs, edge_index, adj_vals, w_init_0, w_x_0, w_init_1, w_x_1,
           w_init_2, w_x_2, w_init_3, w_x_3):
    src = edge_index[0]
    dst = edge_index[1]
    pad = EP - E
    pidx = jnp.arange(pad, dtype=jnp.int32) % N
    srcp = jnp.concatenate([src, pidx])
    dstp = jnp.concatenate([dst, pidx])
    valp = jnp.concatenate([adj_vals, jnp.zeros((pad,), jnp.float32)])
    src2 = jnp.concatenate([srcp, srcp + N]).reshape(2 * ER, 128)
    dst2 = dstp.reshape(ER, 128)
    val2 = valp.reshape(ER, 128)

    x0p = jnp.pad(inputs, ((0, 0), (0, 13)))
    wi0p = jnp.pad(w_init_0, ((0, 13), (0, 13)))
    wx0p = jnp.pad(w_x_0, ((0, 13), (0, 0)))
    wi1p = jnp.pad(w_init_1, ((0, 13), (0, 0)))
    wi2p = jnp.pad(w_init_2, ((0, 13), (0, 0)))
    wi3p = jnp.pad(w_init_3, ((0, 13), (0, 0)))
    wx3p = jnp.pad(w_x_3, ((0, 0), (0, 15)))

    g0 = _spmm16(x0p, src2, dst2, val2)
    x1 = _transform0(g0, x0p, wi0p, wx0p).reshape(2 * N, 16)
    g1 = _spmm32(x1, src2, dst2, val2)
    x2 = _transform_mid(g1, x0p, wi1p, w_x_1).reshape(2 * N, 16)
    g2 = _spmm32(x2, src2, dst2, val2)
    y3 = _transform_y(g2, x0p, wi2p, w_x_2, wx3p)
    g3 = _spmm16(y3, src2, dst2, val2)
    return _transform_last(g3, x0p, wi3p, wx3p)


# speedup vs baseline: 1.9945x; 1.0115x over previous
import functools

import jax
import jax.numpy as jnp
from jax import lax
from jax.experimental import pallas as pl
from jax.experimental.pallas import tpu as pltpu
from jax.experimental.pallas import tpu_sc as plsc

N = 100000
E = 3200000
EP = 3276800
ER = EP // 128
ALPHA = 0.9
KR = 8
NP = 100352
TPT = NP // 16
ZROWS = 112
ZREP = TPT // ZROWS


def _make_spmm(split_features: bool, xt_rows: int):
    mesh = plsc.VectorSubcoreMesh(core_axis_name="c", subcore_axis_name="s")
    DEPTH = 6

    @functools.partial(
        pl.kernel,
        out_type=jax.ShapeDtypeStruct((2, NP, 16), jnp.float32),
        mesh=mesh,
        compiler_params=pltpu.CompilerParams(use_tc_tiling_on_sc=False),
        scratch_types=[
            pltpu.VMEM((4, KR, 128), jnp.int32),
            pltpu.VMEM((4, KR, 128), jnp.int32),
            pltpu.VMEM((4, KR, 128), jnp.float32),
            pltpu.VMEM((KR * 128, 16), jnp.float32),
            pltpu.VMEM((ZROWS, 16), jnp.float32),
            pltpu.VMEM_SHARED((NP, 16), jnp.float32),
            pltpu.SemaphoreType.DMA((4,)),
            pltpu.SemaphoreType.DMA((KR,)),
            pltpu.SemaphoreType.DMA((KR,)),
        ],
    )
    def spmm(xt, srcr, dstr, valr, out, sbuf, dbuf, vbuf, rows, zbuf, acc,
             isem, gsem, ssem):
        c = lax.axis_index("c")
        s = lax.axis_index("s")

        with jax.named_scope("acc_zero"):
            zeros16 = jnp.zeros((16,), jnp.float32)

            def zb(i, carry):
                zbuf[i, :] = zeros16
                return carry

            lax.fori_loop(0, ZROWS, zb, 0)
            zb0 = s * TPT

            def zcopy(z, carry):
                pltpu.sync_copy(zbuf, acc.at[pl.ds(zb0 + z * ZROWS, ZROWS)])
                return carry

            lax.fori_loop(0, ZREP, zcopy, 0)
            plsc.subcore_barrier()

        if split_features:
            sbase = c * ER + s * (ER // 16)
            dbase = s * (ER // 16)
            ngroups = ER // 16
        else:
            w = s * 2 + c
            sbase = w * (ER // 32)
            dbase = sbase
            ngroups = ER // 32
        nmac = ngroups // KR

        def stage_idx(k, slot):
            pltpu.async_copy(srcr.at[pl.ds(sbase + k * KR, KR)],
                             sbuf.at[slot], isem.at[slot])
            pltpu.async_copy(dstr.at[pl.ds(dbase + k * KR, KR)],
                             dbuf.at[slot], isem.at[slot])
            pltpu.async_copy(valr.at[pl.ds(dbase + k * KR, KR)],
                             vbuf.at[slot], isem.at[slot])

        def wait_idx(k, slot):
            pltpu.make_async_copy(srcr.at[pl.ds(sbase + k * KR, KR)],
                                  sbuf.at[slot], isem.at[slot]).wait()
            pltpu.make_async_copy(dstr.at[pl.ds(dbase + k * KR, KR)],
                                  dbuf.at[slot], isem.at[slot]).wait()
            pltpu.make_async_copy(valr.at[pl.ds(dbase + k * KR, KR)],
                                  vbuf.at[slot], isem.at[slot]).wait()

        def fire_gather(f):
            kf = lax.rem(lax.div(f, KR), 4)
            fr = lax.rem(f, KR)
            pltpu.async_copy(xt.at[sbuf.at[kf, fr]],
                             rows.at[pl.ds(fr * 128, 128)], gsem.at[fr])

        stage_idx(0, 0)
        stage_idx(1, 1)
        stage_idx(2, 2)
        wait_idx(0, 0)
        for g in range(DEPTH):
            fire_gather(g)

        def step(g, carry):
            r = lax.rem(g, KR)
            km = lax.rem(lax.div(g, KR), 4)
            rj = rows.at[pl.ds(r * 128, 128)]
            pltpu.make_async_copy(xt.at[sbuf.at[km, r]], rj,
                                  gsem.at[r]).wait()
            for q in range(8):
                vch = vbuf[km, r, pl.ds(q * 16, 16)]
                for l in range(16):
                    b = lax.gather(
                        vch, jnp.full((16, 1), l, jnp.int32),
                        lax.GatherDimensionNumbers(
                            offset_dims=(), collapsed_slice_dims=(0,),
                            start_index_map=(0,)),
                        (1,), mode=lax.GatherScatterMode.PROMISE_IN_BOUNDS)
                    rr = r * 128 + q * 16 + l
                    rows[rr, :] = rows[rr, :] * b
            pltpu.async_copy(rj, acc.at[dbuf.at[km, r]], ssem.at[r],
                             add=True)

            f = g + DEPTH

            @pl.when(f < ngroups)
            def _ahead():
                kf = lax.div(f, KR)
                fr = lax.rem(f, KR)

                @pl.when(lax.rem(f, KR) == 0)
                def _boundary():
                    @pl.when(kf + 2 < nmac)
                    def _stage():
                        stage_idx(kf + 2, lax.rem(kf + 2, 4))

                    wait_idx(kf, lax.rem(kf, 4))

                @pl.when(f >= KR)
                def _wait_prev_scatter():
                    kp = lax.rem(lax.div(f, KR) + 3, 4)
                    pltpu.make_async_copy(
                        rows.at[pl.ds(fr * 128, 128)],
                        acc.at[dbuf.at[kp, fr]], ssem.at[fr]).wait()

                fire_gather(f)

            return carry

        with jax.named_scope("edge_loop"):
            lax.fori_loop(0, ngroups, step, 0)

            klast = (nmac - 1) % 4
            for g in range(KR):
                pltpu.make_async_copy(rows.at[pl.ds(g * 128, 128)],
                                      acc.at[dbuf.at[klast, g]],
                                      ssem.at[g]).wait()

        with jax.named_scope("writeback"):
            plsc.subcore_barrier()
            wb0 = s * TPT
            pltpu.sync_copy(acc.at[pl.ds(wb0, TPT)],
                            out.at[c, pl.ds(wb0, TPT)])

    return spmm


_spmm16 = _make_spmm(split_features=False, xt_rows=N)
_spmm32 = _make_spmm(split_features=True, xt_rows=2 * N)

_B = 4000
_NB = N // _B


def _dot(a, b):
    return jnp.dot(a, b, preferred_element_type=jnp.float32,
                   precision=lax.Precision.HIGHEST)


def _transform0(g, x0p, wi0p, wx0p):
    def body(g0, g1, x0, wi, wx, out):
        u = ALPHA * (g0[0] + g1[0]) + (1.0 - ALPHA) * _dot(x0[...], wi[...])
        x1 = jnp.maximum(_dot(u, wx[...]), 0.0)
        out[0, :, :] = x1[:, :16]
        out[1, :, :] = x1[:, 16:]

    return pl.pallas_call(
        body,
        grid=(_NB,),
        in_specs=[
            pl.BlockSpec((1, _B, 16), lambda i: (0, i, 0)),
            pl.BlockSpec((1, _B, 16), lambda i: (1, i, 0)),
            pl.BlockSpec((_B, 16), lambda i: (i, 0)),
            pl.BlockSpec((16, 16), lambda i: (0, 0)),
            pl.BlockSpec((16, 32), lambda i: (0, 0)),
        ],
        out_specs=pl.BlockSpec((2, _B, 16), lambda i: (0, i, 0)),
        out_shape=jax.ShapeDtypeStruct((2, N, 16), jnp.float32),
    )(g, g, x0p, wi0p, wx0p)


def _transform_mid(g, x0p, wip, wx):
    def body(g0, g1, x0, wi, wxr, out):
        gcat = jnp.concatenate([g0[0], g1[0]], axis=1)
        u = ALPHA * gcat + (1.0 - ALPHA) * _dot(x0[...], wi[...])
        x = jnp.maximum(_dot(u, wxr[...]), 0.0)
        out[0, :, :] = x[:, :16]
        out[1, :, :] = x[:, 16:]

    return pl.pallas_call(
        body,
        grid=(_NB,),
        in_specs=[
            pl.BlockSpec((1, _B, 16), lambda i: (0, i, 0)),
            pl.BlockSpec((1, _B, 16), lambda i: (1, i, 0)),
            pl.BlockSpec((_B, 16), lambda i: (i, 0)),
            pl.BlockSpec((16, 32), lambda i: (0, 0)),
            pl.BlockSpec((32, 32), lambda i: (0, 0)),
        ],
        out_specs=pl.BlockSpec((2, _B, 16), lambda i: (0, i, 0)),
        out_shape=jax.ShapeDtypeStruct((2, N, 16), jnp.float32),
    )(g, g, x0p, wip, wx)


def _transform_y(g, x0p, wip, wx, wx3p):
    def body(g0, g1, x0, wi, wxr, wx3, out):
        gcat = jnp.concatenate([g0[0], g1[0]], axis=1)
        u = ALPHA * gcat + (1.0 - ALPHA) * _dot(x0[...], wi[...])
        x = jnp.maximum(_dot(u, wxr[...]), 0.0)
        out[...] = _dot(x, wx3[...])

    return pl.pallas_call(
        body,
        grid=(_NB,),
        in_specs=[
            pl.BlockSpec((1, _B, 16), lambda i: (0, i, 0)),
            pl.BlockSpec((1, _B, 16), lambda i: (1, i, 0)),
            pl.BlockSpec((_B, 16), lambda i: (i, 0)),
            pl.BlockSpec((16, 32), lambda i: (0, 0)),
            pl.BlockSpec((32, 32), lambda i: (0, 0)),
            pl.BlockSpec((32, 16), lambda i: (0, 0)),
        ],
        out_specs=pl.BlockSpec((_B, 16), lambda i: (i, 0)),
        out_shape=jax.ShapeDtypeStruct((N, 16), jnp.float32),
    )(g, g, x0p, wip, wx, wx3p)


def _transform_last(g, x0p, wi3p, wx3p):
    def body(g0, g1, x0, wi, wx, out):
        c3 = _dot(wi[...], wx[...])
        t = _dot(x0[...], c3)
        v = ALPHA * (g0[0] + g1[0]) + (1.0 - ALPHA) * t
        vc = v[:, 0:1]
        out[...] = 1.0 / (1.0 + jnp.exp(-vc))

    return pl.pallas_call(
        body,
        grid=(_NB,),
        in_specs=[
            pl.BlockSpec((1, _B, 16), lambda i: (0, i, 0)),
            pl.BlockSpec((1, _B, 16), lambda i: (1, i, 0)),
            pl.BlockSpec((_B, 16), lambda i: (i, 0)),
            pl.BlockSpec((16, 32), lambda i: (0, 0)),
            pl.BlockSpec((32, 16), lambda i: (0, 0)),
        ],
        out_specs=pl.BlockSpec((_B, 1), lambda i: (i, 0)),
        out_shape=jax.ShapeDtypeStruct((N, 1), jnp.float32),
    )(g, g, x0p, wi3p, wx3p)


def kernel(inputs, edge_index, adj_vals, w_init_0, w_x_0, w_init_1, w_x_1,
           w_init_2, w_x_2, w_init_3, w_x_3):
    src = edge_index[0]
    dst = edge_index[1]
    pad = EP - E
    pidx = jnp.arange(pad, dtype=jnp.int32) % N
    srcp = jnp.concatenate([src, pidx])
    dstp = jnp.concatenate([dst, pidx])
    valp = jnp.concatenate([adj_vals, jnp.zeros((pad,), jnp.float32)])
    src2 = jnp.concatenate([srcp, srcp + N]).reshape(2 * ER, 128)
    dst2 = dstp.reshape(ER, 128)
    val2 = valp.reshape(ER, 128)

    x0p = jnp.pad(inputs, ((0, 0), (0, 13)))
    wi0p = jnp.pad(w_init_0, ((0, 13), (0, 13)))
    wx0p = jnp.pad(w_x_0, ((0, 13), (0, 0)))
    wi1p = jnp.pad(w_init_1, ((0, 13), (0, 0)))
    wi2p = jnp.pad(w_init_2, ((0, 13), (0, 0)))
    wi3p = jnp.pad(w_init_3, ((0, 13), (0, 0)))
    wx3p = jnp.pad(w_x_3, ((0, 0), (0, 15)))

    g0 = _spmm16(x0p, src2, dst2, val2)
    x1 = _transform0(g0, x0p, wi0p, wx0p).reshape(2 * N, 16)
    g1 = _spmm32(x1, src2, dst2, val2)
    x2 = _transform_mid(g1, x0p, wi1p, w_x_1).reshape(2 * N, 16)
    g2 = _spmm32(x2, src2, dst2, val2)
    y3 = _transform_y(g2, x0p, wi2p, w_x_2, wx3p)
    g3 = _spmm16(y3, src2, dst2, val2)
    return _transform_last(g3, x0p, wi3p, wx3p)
